# Initial kernel scaffold; baseline (speedup 1.0000x reference)
import jax
import jax.numpy as jnp
from jax.experimental import pallas as pl


def kernel(sample_vals, sample_posns, query_posns):
    raise NotImplementedError("write your pallas kernel here")


# speedup vs baseline: 1.1972x; 1.1972x over previous
import functools

import jax
import jax.numpy as jnp
from jax import lax
from jax.experimental import pallas as pl
from jax.experimental.pallas import tpu as pltpu
from jax.experimental.pallas import tpu_sc as plsc


def _sumsq(x, axis):
    n = x.shape[axis]
    sl = lambda k: lax.slice_in_dim(x, k, k + 1, axis=axis)
    acc = sl(0) * sl(0)
    for k in range(1, n):
        acc = acc + sl(k) * sl(k)
    return acc


def _nn_idx_body(qp_ref, spt_ref, fidx_ref, *, n_batch):
    b = pl.program_id(0)
    qp = qp_ref[0]
    spt = spt_ref[0]
    ns = spt.shape[1]
    qs = lax.dot_general(qp, spt, (((1,), (0,)), ((), ())))
    q2 = _sumsq(qp, 1)
    s2 = _sumsq(spt, 0)
    d = (q2 + s2) - 2.0 * qs
    mind = jnp.min(d, axis=1, keepdims=True)
    sidx = lax.broadcasted_iota(jnp.int32, d.shape, 1)
    idx = jnp.min(jnp.where(d == mind, sidx, ns), axis=1)
    fidx_ref[0, 0, :] = idx * n_batch + b


def _nn_indices(qp_t, spt, bq):
    n_batch, nq, ndp = qp_t.shape
    ns = spt.shape[2]
    nqb = nq // bq
    fidx = pl.pallas_call(
        functools.partial(_nn_idx_body, n_batch=n_batch),
        grid=(n_batch, nqb),
        in_specs=[
            pl.BlockSpec((1, bq, ndp), lambda b, qi: (b, qi, 0)),
            pl.BlockSpec((1, ndp, ns), lambda b, qi: (b, 0, 0)),
        ],
        out_specs=pl.BlockSpec((1, 1, bq), lambda b, qi: (b * nqb + qi, 0, 0)),
        out_shape=jax.ShapeDtypeStruct((n_batch * nqb, 1, bq), jnp.int32),
    )(qp_t, spt)
    return fidx.reshape(n_batch, nq)


def _sc_gather(table, fidx3):
    nw, kch, lch = fidx3.shape
    rows_per_w = kch * lch
    d = table.shape[1]
    info = plsc.get_sparse_core_info()
    mesh = plsc.VectorSubcoreMesh(core_axis_name="c", subcore_axis_name="s")

    @functools.partial(
        pl.kernel,
        out_type=jax.ShapeDtypeStruct((nw * rows_per_w, d), jnp.float32),
        mesh=mesh,
        scratch_types=[
            pltpu.VMEM((kch, lch), jnp.int32),
            pltpu.VMEM((rows_per_w, d), jnp.float32),
            pltpu.SemaphoreType.DMA,
        ],
    )
    def body(table_hbm, idx_hbm, out_hbm, idx_v, rows_v, sem):
        wid = lax.axis_index("s") * info.num_cores + lax.axis_index("c")
        pltpu.sync_copy(idx_hbm.at[wid], idx_v)
        copies = [
            pltpu.async_copy(
                table_hbm.at[idx_v.at[j]], rows_v.at[pl.ds(j * lch, lch)], sem
            )
            for j in range(kch)
        ]
        for cp in copies:
            cp.wait()
        pltpu.sync_copy(rows_v, out_hbm.at[pl.ds(wid * rows_per_w, rows_per_w)])

    return body(table, fidx3)


def kernel(sample_vals, sample_posns, query_posns):
    ns, n_batch, ndv = sample_vals.shape
    nq = query_posns.shape[0]
    qp_t = jnp.transpose(query_posns, (1, 0, 2))
    spt = jnp.transpose(sample_posns, (1, 2, 0))
    fidx = _nn_indices(qp_t, spt, bq=512)
    fidx_flat = fidx.T.reshape(-1)
    table = sample_vals.reshape(ns * n_batch, ndv)
    info = plsc.get_sparse_core_info()
    nw = info.num_cores * info.num_subcores
    rows_per_w = (nq * n_batch) // nw
    fidx3 = fidx_flat.reshape(nw, rows_per_w // 128, 128)
    out_flat = _sc_gather(table, fidx3)
    return out_flat.reshape(nq, n_batch, ndv)


# speedup vs baseline: 1.5072x; 1.2589x over previous
import functools

import jax
import jax.numpy as jnp
from jax import lax
from jax.experimental import pallas as pl
from jax.experimental.pallas import tpu as pltpu
from jax.experimental.pallas import tpu_sc as plsc


def _sumsq(x, axis):
    n = x.shape[axis]
    sl = lambda k: lax.slice_in_dim(x, k, k + 1, axis=axis)
    acc = sl(0) * sl(0)
    for k in range(1, n):
        acc = acc + sl(k) * sl(k)
    return acc


def _nn_idx_body(qp_ref, spt_ref, iota_ref, fidx_ref, *, n_batch):
    b = pl.program_id(0)
    qp = qp_ref[0]
    spt = spt_ref[0]
    ns = spt.shape[1]
    bq = qp.shape[0]
    qs2 = lax.dot_general(qp + qp, spt, (((1,), (0,)), ((), ())))
    q2 = _sumsq(qp, 1)
    s2 = _sumsq(spt, 0)
    lanef = iota_ref[:, :128]
    rs = 64
    nch = ns // 128
    mins, idxs = [], []
    for r0 in range(0, bq, rs):
        q2b = jnp.broadcast_to(q2[r0:r0 + rs], (rs, 128))
        run_min = (q2b + s2[:, :128]) - qs2[r0:r0 + rs, :128]
        run_idx = jnp.zeros((rs, 128), jnp.float32)
        for c in range(1, nch):
            dch = (q2b + s2[:, c * 128:(c + 1) * 128]) \
                - qs2[r0:r0 + rs, c * 128:(c + 1) * 128]
            won = dch < run_min
            run_min = jnp.minimum(run_min, dch)
            run_idx = jnp.where(won, jnp.float32(c), run_idx)
        mins.append(run_min)
        idxs.append(run_idx)
    run_min = jnp.concatenate(mins, axis=0)
    run_idx = jnp.concatenate(idxs, axis=0)
    gmin = jnp.min(run_min, axis=1, keepdims=True)
    cand = jnp.where(run_min == gmin, run_idx * 128.0 + lanef, 1e9)
    idxf = jnp.min(cand, axis=1)
    fidx_ref[0, 0, :] = idxf.astype(jnp.int32) * n_batch + b


def _nn_indices(qp_t, spt, bq):
    n_batch, nq, ndp = qp_t.shape
    ns = spt.shape[2]
    nqb = nq // bq
    fidx = pl.pallas_call(
        functools.partial(_nn_idx_body, n_batch=n_batch),
        grid=(n_batch, nqb),
        in_specs=[
            pl.BlockSpec((1, bq, ndp), lambda b, qi: (b, qi, 0)),
            pl.BlockSpec((1, ndp, ns), lambda b, qi: (b, 0, 0)),
            pl.BlockSpec((1, ns), lambda b, qi: (0, 0)),
        ],
        out_specs=pl.BlockSpec((1, 1, bq), lambda b, qi: (b * nqb + qi, 0, 0)),
        out_shape=jax.ShapeDtypeStruct((n_batch * nqb, 1, bq), jnp.int32),
    )(qp_t, spt, jnp.arange(ns, dtype=jnp.float32)[None, :])
    return fidx.reshape(n_batch, nq)


def _sc_gather(table, fidx3):
    nw, kch, lch = fidx3.shape
    rows_per_w = kch * lch
    d = table.shape[1]
    info = plsc.get_sparse_core_info()
    mesh = plsc.VectorSubcoreMesh(core_axis_name="c", subcore_axis_name="s")

    @functools.partial(
        pl.kernel,
        out_type=jax.ShapeDtypeStruct((nw * rows_per_w, d), jnp.float32),
        mesh=mesh,
        scratch_types=[
            pltpu.VMEM((kch, lch), jnp.int32),
            pltpu.VMEM((rows_per_w, d), jnp.float32),
            pltpu.SemaphoreType.DMA,
        ],
    )
    def body(table_hbm, idx_hbm, out_hbm, idx_v, rows_v, sem):
        wid = lax.axis_index("s") * info.num_cores + lax.axis_index("c")
        pltpu.sync_copy(idx_hbm.at[wid], idx_v)
        copies = [
            pltpu.async_copy(
                table_hbm.at[idx_v.at[j]], rows_v.at[pl.ds(j * lch, lch)], sem
            )
            for j in range(kch)
        ]
        for cp in copies:
            cp.wait()
        pltpu.sync_copy(rows_v, out_hbm.at[pl.ds(wid * rows_per_w, rows_per_w)])

    return body(table, fidx3)


def kernel(sample_vals, sample_posns, query_posns):
    ns, n_batch, ndv = sample_vals.shape
    nq = query_posns.shape[0]
    qp_t = jnp.transpose(query_posns, (1, 0, 2))
    spt = jnp.transpose(sample_posns, (1, 2, 0))
    fidx = _nn_indices(qp_t, spt, bq=512)
    fidx_flat = fidx.T.reshape(-1)
    table = sample_vals.reshape(ns * n_batch, ndv)
    info = plsc.get_sparse_core_info()
    nw = info.num_cores * info.num_subcores
    rows_per_w = (nq * n_batch) // nw
    fidx3 = fidx_flat.reshape(nw, rows_per_w // 128, 128)
    out_flat = _sc_gather(table, fidx3)
    return out_flat.reshape(nq, n_batch, ndv)


# speedup vs baseline: 1.7352x; 1.1512x over previous
import functools

import jax
import jax.numpy as jnp
from jax import lax
from jax.experimental import pallas as pl
from jax.experimental.pallas import tpu as pltpu
from jax.experimental.pallas import tpu_sc as plsc


def _sumsq(x, axis):
    n = x.shape[axis]
    sl = lambda k: lax.slice_in_dim(x, k, k + 1, axis=axis)
    acc = sl(0) * sl(0)
    for k in range(1, n):
        acc = acc + sl(k) * sl(k)
    return acc


def _nn_idx_body(qp_ref, spt_ref, iota_ref, fidx_ref, *, ndp):
    qp12 = qp_ref[...]
    lanef = iota_ref[...]
    n_batch = spt_ref.shape[0]
    ns = spt_ref.shape[2]
    bq = qp12.shape[0]
    rs = 64
    nch = ns // 128
    for b in range(n_batch):
        qp = qp12[:, b * ndp:(b + 1) * ndp]
        spt = spt_ref[b]
        qs2 = lax.dot_general(qp + qp, spt, (((1,), (0,)), ((), ())))
        q2 = _sumsq(qp, 1)
        s2 = _sumsq(spt, 0)
        mins, idxs = [], []
        for r0 in range(0, bq, rs):
            q2b = jnp.broadcast_to(q2[r0:r0 + rs], (rs, 128))
            run_min = (q2b + s2[:, :128]) - qs2[r0:r0 + rs, :128]
            run_idx = jnp.zeros((rs, 128), jnp.float32)
            for c in range(1, nch):
                dch = (q2b + s2[:, c * 128:(c + 1) * 128]) \
                    - qs2[r0:r0 + rs, c * 128:(c + 1) * 128]
                won = dch < run_min
                run_min = jnp.minimum(run_min, dch)
                run_idx = jnp.where(won, jnp.float32(c), run_idx)
            mins.append(run_min)
            idxs.append(run_idx)
        run_min = jnp.concatenate(mins, axis=0)
        run_idx = jnp.concatenate(idxs, axis=0)
        gmin = jnp.min(run_min, axis=1, keepdims=True)
        cand = jnp.where(run_min == gmin, run_idx * 128.0 + lanef, 1e9)
        idxf = jnp.min(cand, axis=1)
        fidx_ref[0, b, 0, :] = idxf.astype(jnp.int32) * n_batch + b


def _nn_indices(qp12, spt, bq):
    nq = qp12.shape[0]
    n_batch, ndp, ns = spt.shape
    nqb = nq // bq
    fidx = pl.pallas_call(
        functools.partial(_nn_idx_body, ndp=ndp),
        grid=(nqb,),
        in_specs=[
            pl.BlockSpec((bq, n_batch * ndp), lambda qi: (qi, 0)),
            pl.BlockSpec((n_batch, ndp, ns), lambda qi: (0, 0, 0)),
            pl.BlockSpec((1, 128), lambda qi: (0, 0)),
        ],
        out_specs=pl.BlockSpec((1, n_batch, 1, bq), lambda qi: (qi, 0, 0, 0)),
        out_shape=jax.ShapeDtypeStruct((nqb, n_batch, 1, bq), jnp.int32),
    )(qp12, spt, jnp.arange(128, dtype=jnp.float32)[None, :])
    return fidx


def _sc_gather(table, fidx4, nq, n_batch):
    nqb, _, kch, lch = fidx4.shape
    rows_per_w = kch * lch
    ndv = table.shape[1]
    info = plsc.get_sparse_core_info()
    mesh = plsc.VectorSubcoreMesh(core_axis_name="c", subcore_axis_name="s")

    @functools.partial(
        pl.kernel,
        out_type=jax.ShapeDtypeStruct((nq, n_batch, ndv), jnp.float32),
        mesh=mesh,
        scratch_types=[
            pltpu.VMEM((kch, lch), jnp.int32),
            pltpu.VMEM((rows_per_w, ndv), jnp.float32),
            pltpu.SemaphoreType.DMA,
        ],
    )
    def body(table_hbm, idx_hbm, out_hbm, idx_v, rows_v, sem):
        wid = lax.axis_index("s") * info.num_cores + lax.axis_index("c")
        qi = wid // n_batch
        b = wid % n_batch
        pltpu.sync_copy(idx_hbm.at[qi, b], idx_v)
        copies = [
            pltpu.async_copy(
                table_hbm.at[idx_v.at[j]], rows_v.at[pl.ds(j * lch, lch)], sem
            )
            for j in range(kch)
        ]
        for cp in copies:
            cp.wait()
        pltpu.sync_copy(rows_v, out_hbm.at[pl.ds(qi * rows_per_w, rows_per_w), b])

    return body(table, fidx4)


def kernel(sample_vals, sample_posns, query_posns):
    ns, n_batch, ndv = sample_vals.shape
    nq, _, ndp = query_posns.shape
    qp12 = query_posns.reshape(nq, n_batch * ndp)
    spt = jnp.transpose(sample_posns, (1, 2, 0))
    bq = 512
    fidx = _nn_indices(qp12, spt, bq)
    table = sample_vals.reshape(ns * n_batch, ndv)
    fidx4 = fidx.reshape(nq // bq, n_batch, bq // 128, 128)
    return _sc_gather(table, fidx4, nq, n_batch)
